# baseline (speedup 1.0000x reference)
import jax
import jax.numpy as jnp
from jax import lax
from jax.experimental import pallas as pl
from jax.experimental.pallas import tpu as pltpu
from jax.experimental.pallas import tpu_sc as plsc

N_NODES = 10000
N_EDGES = 320000
D = 128
NEG_SLOPE = 0.2

NC = 2
NS = 16
NW = NC * NS
CHUNK = 80
NCH = N_EDGES // (NW * CHUNK)
NSA = 5
ECA = NCH // NSA
SBB = 5
NST = NCH // SBB
SUP = 15
NOUT = (NCH - SBB) // SUP
ACH = N_NODES // CHUNK
DEN_R = 80


def _tc_pre_body(x_ref, w_ref, ap_ref, msg_ref, al_ref):
    msg = jnp.dot(x_ref[...], w_ref[...], preferred_element_type=jnp.float32)
    msg_ref[...] = msg
    al_ref[...] = jnp.dot(msg, ap_ref[...], preferred_element_type=jnp.float32)


def _tc_pre(x, w, a_pad):
    blk = 1000
    grid = N_NODES // blk
    return pl.pallas_call(
        _tc_pre_body,
        grid=(grid,),
        in_specs=[
            pl.BlockSpec((blk, D), lambda i: (i, 0)),
            pl.BlockSpec((D, D), lambda i: (0, 0)),
            pl.BlockSpec((D, D), lambda i: (0, 0)),
        ],
        out_specs=[
            pl.BlockSpec((blk, D), lambda i: (i, 0)),
            pl.BlockSpec((blk, D), lambda i: (i, 0)),
        ],
        out_shape=[
            jax.ShapeDtypeStruct((N_NODES, D), jnp.float32),
            jax.ShapeDtypeStruct((N_NODES, D), jnp.float32),
        ],
    )(x, w, a_pad)


def _sc_a_body(row_h, col_h, nv_h, asrc_h, adst_h, w_h, den_h,
               rowa, cola, nva, wbuf, asrc_v, adst_v, denb):
    cid = lax.axis_index("c")
    sid = lax.axis_index("s")
    wid = cid * NS + sid

    pltpu.sync_copy(asrc_h, asrc_v)
    pltpu.sync_copy(adst_h, adst_v)

    z16 = jnp.zeros((16,), jnp.float32)

    def zden(r, _):
        for k in range(D // 16):
            denb[r, pl.ds(k * 16, 16)] = z16
        return 0

    lax.fori_loop(0, DEN_R, zden, 0)

    def stage(s, _):
        pltpu.sync_copy(row_h.at[wid, s], rowa)
        pltpu.sync_copy(col_h.at[wid, s], cola)
        pltpu.sync_copy(nv_h.at[wid, s], nva)

        def chunk(j, _):
            for g in range(CHUNK // 16):
                r16 = rowa[j, pl.ds(g * 16, 16)]
                c16 = cola[j, pl.ds(g * 16, 16)]
                logit = (plsc.load_gather(asrc_v, [r16]) +
                         plsc.load_gather(adst_v, [c16]))
                logit = jnp.where(logit >= 0, logit, NEG_SLOPE * logit)
                ex = jnp.exp(logit)
                wbuf[j, pl.ds(g * 16, 16)] = nva[j, pl.ds(g * 16, 16)] * ex
                hi = lax.shift_right_logical(r16, 7)
                lo = jnp.bitwise_and(r16, 127)
                plsc.addupdate_scatter(denb, [hi, lo], ex)
            return 0

        lax.fori_loop(0, ECA, chunk, 0)
        pltpu.sync_copy(wbuf, w_h.at[wid, s])
        return 0

    lax.fori_loop(0, NSA, stage, 0)
    pltpu.sync_copy(denb, den_h.at[cid, sid])


def _sc_a(row4d, col4d, nv4d, a_src, a_dst):
    mesh = plsc.VectorSubcoreMesh(core_axis_name="c", subcore_axis_name="s")
    f = pl.kernel(
        _sc_a_body,
        out_type=[
            jax.ShapeDtypeStruct((NW, NSA, ECA, CHUNK), jnp.float32),
            jax.ShapeDtypeStruct((NC, NS, DEN_R, D), jnp.float32),
        ],
        mesh=mesh,
        scratch_types=[
            pltpu.VMEM((ECA, CHUNK), jnp.int32),
            pltpu.VMEM((ECA, CHUNK), jnp.int32),
            pltpu.VMEM((ECA, CHUNK), jnp.float32),
            pltpu.VMEM((ECA, CHUNK), jnp.float32),
            pltpu.VMEM((N_NODES,), jnp.float32),
            pltpu.VMEM((N_NODES,), jnp.float32),
            pltpu.VMEM((DEN_R, D), jnp.float32),
        ],
        compiler_params=pltpu.CompilerParams(needs_layout_passes=False),
    )
    return f(row4d, col4d, nv4d, a_src, a_dst)


def _sc_b_body(row_h, col_h, w_h, msg_h, num_h,
               rows0, rows1, rows2,
               ecr0, ecr1, ecr2, ecc0, ecc1, ecc2, ecw0, ecw1, ecw2,
               w_v, acc_sh, g0, g1, g2, s0, s1, s2, esem):
    cid = lax.axis_index("c")
    sid = lax.axis_index("s")
    wid = cid * NS + sid
    bufs = [rows0, rows1, rows2]
    ecrs = [ecr0, ecr1, ecr2]
    eccs = [ecc0, ecc1, ecc2]
    ecws = [ecw0, ecw1, ecw2]
    gsems = [g0, g1, g2]
    ssems = [s0, s1, s2]

    z16 = jnp.zeros((16,), jnp.float32)

    def zr(r, _):
        for k in range(D // 16):
            rows0[r, pl.ds(k * 16, 16)] = z16
        return 0

    lax.fori_loop(0, CHUNK, zr, 0)
    for k in range(ACH // NS + 1):
        t = sid + NS * k

        @pl.when(t < ACH)
        def _():
            pltpu.sync_copy(rows0, acc_sh.at[pl.ds(t * CHUNK, CHUNK)])

    plsc.subcore_barrier()

    for sl in (0, 1):
        pltpu.sync_copy(row_h.at[wid, sl], ecrs[sl])
        pltpu.sync_copy(col_h.at[wid, sl], eccs[sl])
        pltpu.sync_copy(w_h.at[wid, sl], ecws[sl])

    pltpu.async_copy(msg_h.at[eccs[0].at[0]], bufs[0], gsems[0])
    pltpu.async_copy(msg_h.at[eccs[0].at[1]], bufs[1], gsems[1])

    def step(p, t):
        u = p * SUP + t
        b = t % 3
        sl = (t // SBB) % 3
        jj = t % SBB

        pltpu.make_async_copy(
            msg_h.at[pl.ds(0, CHUNK)], bufs[b], gsems[b]).wait()

        u2 = u + 2
        t2 = t + 2
        b2 = t2 % 3
        sl2 = (t2 // SBB) % 3
        jj2 = t2 % SBB

        @pl.when(u2 < NCH)
        def _():
            @pl.when(u >= 1)
            def _():
                pltpu.make_async_copy(
                    bufs[b2], acc_sh.at[pl.ds(0, CHUNK)], ssems[b2]).wait()

            if jj2 == 0:
                @pl.when(u2 >= 2 * SBB)
                def _():
                    pltpu.make_async_copy(
                        row_h.at[wid, 0], ecrs[sl2], esem).wait()
                    pltpu.make_async_copy(
                        col_h.at[wid, 0], eccs[sl2], esem).wait()
                    pltpu.make_async_copy(
                        w_h.at[wid, 0], ecws[sl2], esem).wait()

            if jj2 == 3:
                sn = lax.div(u2, SBB) + 2
                sln = (sl2 + 2) % 3

                @pl.when(sn < NST)
                def _():
                    pltpu.async_copy(row_h.at[wid, sn], ecrs[sln], esem)
                    pltpu.async_copy(col_h.at[wid, sn], eccs[sln], esem)
                    pltpu.async_copy(w_h.at[wid, sn], ecws[sln], esem)

            pltpu.async_copy(msg_h.at[eccs[sl2].at[jj2]], bufs[b2], gsems[b2])

        for g in range(CHUNK // 16):
            w_v[pl.ds(g * 16, 16)] = ecws[sl][jj, pl.ds(g * 16, 16)]

        def edge4(q, _):
            for dd in range(4):
                i = q * 4 + dd
                w = plsc.load_gather(w_v, [jnp.full((16,), i, jnp.int32)])
                for kk in range(D // 16):
                    bufs[b][i, pl.ds(kk * 16, 16)] = (
                        bufs[b][i, pl.ds(kk * 16, 16)] * w)
            return 0

        lax.fori_loop(0, CHUNK // 4, edge4, 0)

        pltpu.async_copy(bufs[b], acc_sh.at[ecrs[sl].at[jj]], ssems[b],
                         add=True)

    def outer(p, _):
        for t in range(SUP):
            step(p, t)
        return 0

    lax.fori_loop(0, NOUT, outer, 0)
    for t in range(SBB):
        step(jnp.int32(NOUT), t)

    for b in (0, 1, 2):
        pltpu.make_async_copy(
            bufs[b], acc_sh.at[pl.ds(0, CHUNK)], ssems[b]).wait()

    plsc.subcore_barrier()
    for k in range(ACH // NS + 1):
        t = sid + NS * k

        @pl.when(t < ACH)
        def _():
            pltpu.sync_copy(acc_sh.at[pl.ds(t * CHUNK, CHUNK)],
                            num_h.at[cid, pl.ds(t * CHUNK, CHUNK)])


def _sc_b(row4d, col4d, w4d, msg):
    mesh = plsc.VectorSubcoreMesh(core_axis_name="c", subcore_axis_name="s")
    f = pl.kernel(
        _sc_b_body,
        out_type=jax.ShapeDtypeStruct((NC, N_NODES, D), jnp.float32),
        mesh=mesh,
        scratch_types=[
            pltpu.VMEM((CHUNK, D), jnp.float32),
            pltpu.VMEM((CHUNK, D), jnp.float32),
            pltpu.VMEM((CHUNK, D), jnp.float32),
            pltpu.VMEM((SBB, CHUNK), jnp.int32),
            pltpu.VMEM((SBB, CHUNK), jnp.int32),
            pltpu.VMEM((SBB, CHUNK), jnp.int32),
            pltpu.VMEM((SBB, CHUNK), jnp.int32),
            pltpu.VMEM((SBB, CHUNK), jnp.int32),
            pltpu.VMEM((SBB, CHUNK), jnp.int32),
            pltpu.VMEM((SBB, CHUNK), jnp.float32),
            pltpu.VMEM((SBB, CHUNK), jnp.float32),
            pltpu.VMEM((SBB, CHUNK), jnp.float32),
            pltpu.VMEM((CHUNK,), jnp.float32),
            pltpu.VMEM_SHARED((N_NODES, D), jnp.float32),
            pltpu.SemaphoreType.DMA,
            pltpu.SemaphoreType.DMA,
            pltpu.SemaphoreType.DMA,
            pltpu.SemaphoreType.DMA,
            pltpu.SemaphoreType.DMA,
            pltpu.SemaphoreType.DMA,
            pltpu.SemaphoreType.DMA,
        ],
        compiler_params=pltpu.CompilerParams(needs_layout_passes=False),
    )
    return f(row4d, col4d, w4d, msg)


def _tc_fin_body(np_ref, dp_ref, out_ref):
    n0 = np_ref[0]
    n1 = np_ref[1]
    num = n0 + n1
    den = jnp.sum(dp_ref[...], axis=1)[:, None]
    safe = den > 0
    inv = jnp.where(safe, 1.0 / jnp.where(safe, den, 1.0), 0.0)
    out_ref[...] = num * inv


def _tc_fin(num_part, den_part):
    blk = 1000
    grid = N_NODES // blk
    return pl.pallas_call(
        _tc_fin_body,
        grid=(grid,),
        in_specs=[
            pl.BlockSpec((NC, blk, D), lambda i: (0, i, 0)),
            pl.BlockSpec((blk, NW), lambda i: (i, 0)),
        ],
        out_specs=pl.BlockSpec((blk, D), lambda i: (i, 0)),
        out_shape=jax.ShapeDtypeStruct((N_NODES, D), jnp.float32),
    )(num_part, den_part)


@jax.jit
def kernel(x, edge_index, neighborhood_values, W, a):
    row4d = edge_index[0].reshape(NW, NSA, ECA, CHUNK)
    col4d = edge_index[1].reshape(NW, NSA, ECA, CHUNK)
    nv4d = neighborhood_values.reshape(NW, NSA, ECA, CHUNK)
    a_pad = jnp.zeros((D, D), jnp.float32)
    a_pad = a_pad.at[:, 0].set(a[:D, 0]).at[:, 1].set(a[D:, 0])

    msg, alphas = _tc_pre(x, W, a_pad)
    a_src = alphas[:, 0]
    a_dst = alphas[:, 1]

    w4d, den_part = _sc_a(row4d, col4d, nv4d, a_src, a_dst)
    row4b = edge_index[0].reshape(NW, NST, SBB, CHUNK)
    col4b = edge_index[1].reshape(NW, NST, SBB, CHUNK)
    w4b = w4d.reshape(NW, NST, SBB, CHUNK)
    num_part = _sc_b(row4b, col4b, w4b, msg)

    den2d = den_part.reshape(NW, DEN_R * D)[:, :N_NODES].T
    return _tc_fin(num_part, den2d)


# speedup vs baseline: 1.1920x; 1.1920x over previous
import jax
import jax.numpy as jnp
from jax import lax
from jax.experimental import pallas as pl
from jax.experimental.pallas import tpu as pltpu
from jax.experimental.pallas import tpu_sc as plsc

N_NODES = 10000
N_EDGES = 320000
D = 128
NEG_SLOPE = 0.2

NC = 2
NS = 16
NW = NC * NS
CHUNK = 80
NCH = N_EDGES // (NW * CHUNK)
NSA = 5
ECA = NCH // NSA
SBB = 5
NST = NCH // SBB
SUP = 20
NOUT = (NCH - SBB) // SUP
ACH = N_NODES // CHUNK
DEN_R = 80


def _tc_pre_body(x_ref, w_ref, ap_ref, msg_ref, al_ref):
    msg = jnp.dot(x_ref[...], w_ref[...], preferred_element_type=jnp.float32)
    msg_ref[...] = msg
    al_ref[...] = jnp.dot(msg, ap_ref[...], preferred_element_type=jnp.float32)


def _tc_pre(x, w, a_pad):
    blk = 1000
    grid = N_NODES // blk
    return pl.pallas_call(
        _tc_pre_body,
        grid=(grid,),
        in_specs=[
            pl.BlockSpec((blk, D), lambda i: (i, 0)),
            pl.BlockSpec((D, D), lambda i: (0, 0)),
            pl.BlockSpec((D, D), lambda i: (0, 0)),
        ],
        out_specs=[
            pl.BlockSpec((blk, D), lambda i: (i, 0)),
            pl.BlockSpec((blk, D), lambda i: (i, 0)),
        ],
        out_shape=[
            jax.ShapeDtypeStruct((N_NODES, D), jnp.float32),
            jax.ShapeDtypeStruct((N_NODES, D), jnp.float32),
        ],
    )(x, w, a_pad)


def _sc_a_body(row_h, col_h, nv_h, asrc_h, adst_h, w_h, den_h,
               rowa, cola, nva, wbuf, asrc_v, adst_v, denb):
    cid = lax.axis_index("c")
    sid = lax.axis_index("s")
    wid = cid * NS + sid

    pltpu.sync_copy(asrc_h, asrc_v)
    pltpu.sync_copy(adst_h, adst_v)

    z16 = jnp.zeros((16,), jnp.float32)

    def zden(r, _):
        for k in range(D // 16):
            denb[r, pl.ds(k * 16, 16)] = z16
        return 0

    lax.fori_loop(0, DEN_R, zden, 0)

    def stage(s, _):
        pltpu.sync_copy(row_h.at[wid, s], rowa)
        pltpu.sync_copy(col_h.at[wid, s], cola)
        pltpu.sync_copy(nv_h.at[wid, s], nva)

        def chunk(j, _):
            for g in range(CHUNK // 16):
                r16 = rowa[j, pl.ds(g * 16, 16)]
                c16 = cola[j, pl.ds(g * 16, 16)]
                logit = (plsc.load_gather(asrc_v, [r16]) +
                         plsc.load_gather(adst_v, [c16]))
                logit = jnp.where(logit >= 0, logit, NEG_SLOPE * logit)
                ex = jnp.exp(logit)
                wbuf[j, pl.ds(g * 16, 16)] = nva[j, pl.ds(g * 16, 16)] * ex
                hi = lax.shift_right_logical(r16, 7)
                lo = jnp.bitwise_and(r16, 127)
                plsc.addupdate_scatter(denb, [hi, lo], ex)
            return 0

        lax.fori_loop(0, ECA, chunk, 0)
        pltpu.sync_copy(wbuf, w_h.at[wid, s])
        return 0

    lax.fori_loop(0, NSA, stage, 0)
    pltpu.sync_copy(denb, den_h.at[cid, sid])


def _sc_a(row4d, col4d, nv4d, a_src, a_dst):
    mesh = plsc.VectorSubcoreMesh(core_axis_name="c", subcore_axis_name="s")
    f = pl.kernel(
        _sc_a_body,
        out_type=[
            jax.ShapeDtypeStruct((NW, NSA, ECA, CHUNK), jnp.float32),
            jax.ShapeDtypeStruct((NC, NS, DEN_R, D), jnp.float32),
        ],
        mesh=mesh,
        scratch_types=[
            pltpu.VMEM((ECA, CHUNK), jnp.int32),
            pltpu.VMEM((ECA, CHUNK), jnp.int32),
            pltpu.VMEM((ECA, CHUNK), jnp.float32),
            pltpu.VMEM((ECA, CHUNK), jnp.float32),
            pltpu.VMEM((N_NODES,), jnp.float32),
            pltpu.VMEM((N_NODES,), jnp.float32),
            pltpu.VMEM((DEN_R, D), jnp.float32),
        ],
        compiler_params=pltpu.CompilerParams(needs_layout_passes=False),
    )
    return f(row4d, col4d, nv4d, a_src, a_dst)


def _sc_b_body(row_h, col_h, w_h, msg_h, num_h,
               rows0, rows1, rows2, rows3,
               ecr0, ecr1, ecc0, ecc1, ecw0, ecw1,
               w_v, acc_sh, g0, g1, g2, g3, s0, s1, s2, s3, esem):
    cid = lax.axis_index("c")
    sid = lax.axis_index("s")
    wid = cid * NS + sid
    bufs = [rows0, rows1, rows2, rows3]
    ecrs = [ecr0, ecr1]
    eccs = [ecc0, ecc1]
    ecws = [ecw0, ecw1]
    gsems = [g0, g1, g2, g3]
    ssems = [s0, s1, s2, s3]

    z16 = jnp.zeros((16,), jnp.float32)

    def zr(r, _):
        for k in range(D // 16):
            rows0[r, pl.ds(k * 16, 16)] = z16
        return 0

    lax.fori_loop(0, CHUNK, zr, 0)
    for k in range(ACH // NS + 1):
        t = sid + NS * k

        @pl.when(t < ACH)
        def _():
            pltpu.sync_copy(rows0, acc_sh.at[pl.ds(t * CHUNK, CHUNK)])

    plsc.subcore_barrier()

    pltpu.sync_copy(row_h.at[wid, 0], ecrs[0])
    pltpu.sync_copy(col_h.at[wid, 0], eccs[0])
    pltpu.sync_copy(w_h.at[wid, 0], ecws[0])

    pltpu.async_copy(msg_h.at[eccs[0].at[0]], bufs[0], gsems[0])
    pltpu.async_copy(msg_h.at[eccs[0].at[1]], bufs[1], gsems[1])

    def step(p, t):
        u = p * SUP + t
        b = t % 4
        sl = (t // SBB) % 2
        jj = t % SBB

        pltpu.make_async_copy(
            msg_h.at[pl.ds(0, CHUNK)], bufs[b], gsems[b]).wait()

        u2 = u + 2
        t2 = t + 2
        b2 = t2 % 4
        sl2 = (t2 // SBB) % 2
        jj2 = t2 % SBB

        @pl.when(u2 < NCH)
        def _():
            @pl.when(u >= 2)
            def _():
                pltpu.make_async_copy(
                    bufs[b2], acc_sh.at[pl.ds(0, CHUNK)], ssems[b2]).wait()

            if jj2 == 0:
                @pl.when(u2 >= SBB)
                def _():
                    pltpu.make_async_copy(
                        row_h.at[wid, 0], ecrs[sl2], esem).wait()
                    pltpu.make_async_copy(
                        col_h.at[wid, 0], eccs[sl2], esem).wait()
                    pltpu.make_async_copy(
                        w_h.at[wid, 0], ecws[sl2], esem).wait()

            if jj2 == 3:
                sn = lax.div(u2, SBB) + 1
                sln = (sl2 + 1) % 2

                @pl.when(sn < NST)
                def _():
                    pltpu.async_copy(row_h.at[wid, sn], ecrs[sln], esem)
                    pltpu.async_copy(col_h.at[wid, sn], eccs[sln], esem)
                    pltpu.async_copy(w_h.at[wid, sn], ecws[sln], esem)

            pltpu.async_copy(msg_h.at[eccs[sl2].at[jj2]], bufs[b2], gsems[b2])

        for g in range(CHUNK // 16):
            w_v[pl.ds(g * 16, 16)] = ecws[sl][jj, pl.ds(g * 16, 16)]

        def edge4(q, _):
            for dd in range(4):
                i = q * 4 + dd
                w = plsc.load_gather(w_v, [jnp.full((16,), i, jnp.int32)])
                for kk in range(D // 16):
                    bufs[b][i, pl.ds(kk * 16, 16)] = (
                        bufs[b][i, pl.ds(kk * 16, 16)] * w)
            return 0

        lax.fori_loop(0, CHUNK // 4, edge4, 0)

        pltpu.async_copy(bufs[b], acc_sh.at[ecrs[sl].at[jj]], ssems[b],
                         add=True)

    def outer(p, _):
        for t in range(SUP):
            step(p, t)
        return 0

    lax.fori_loop(0, NOUT, outer, 0)
    for t in range(SBB):
        step(jnp.int32(NOUT), t)

    for b in (1, 2, 3, 0):
        pltpu.make_async_copy(
            bufs[b], acc_sh.at[pl.ds(0, CHUNK)], ssems[b]).wait()

    plsc.subcore_barrier()
    for k in range(ACH // NS + 1):
        t = sid + NS * k

        @pl.when(t < ACH)
        def _():
            pltpu.sync_copy(acc_sh.at[pl.ds(t * CHUNK, CHUNK)],
                            num_h.at[cid, pl.ds(t * CHUNK, CHUNK)])


def _sc_b(row4d, col4d, w4d, msg):
    mesh = plsc.VectorSubcoreMesh(core_axis_name="c", subcore_axis_name="s")
    f = pl.kernel(
        _sc_b_body,
        out_type=jax.ShapeDtypeStruct((NC, N_NODES, D), jnp.float32),
        mesh=mesh,
        scratch_types=[
            pltpu.VMEM((CHUNK, D), jnp.float32),
            pltpu.VMEM((CHUNK, D), jnp.float32),
            pltpu.VMEM((CHUNK, D), jnp.float32),
            pltpu.VMEM((CHUNK, D), jnp.float32),
            pltpu.VMEM((SBB, CHUNK), jnp.int32),
            pltpu.VMEM((SBB, CHUNK), jnp.int32),
            pltpu.VMEM((SBB, CHUNK), jnp.int32),
            pltpu.VMEM((SBB, CHUNK), jnp.int32),
            pltpu.VMEM((SBB, CHUNK), jnp.float32),
            pltpu.VMEM((SBB, CHUNK), jnp.float32),
            pltpu.VMEM((CHUNK,), jnp.float32),
            pltpu.VMEM_SHARED((N_NODES, D), jnp.float32),
            pltpu.SemaphoreType.DMA,
            pltpu.SemaphoreType.DMA,
            pltpu.SemaphoreType.DMA,
            pltpu.SemaphoreType.DMA,
            pltpu.SemaphoreType.DMA,
            pltpu.SemaphoreType.DMA,
            pltpu.SemaphoreType.DMA,
            pltpu.SemaphoreType.DMA,
            pltpu.SemaphoreType.DMA,
        ],
        compiler_params=pltpu.CompilerParams(needs_layout_passes=False),
    )
    return f(row4d, col4d, w4d, msg)


def _tc_fin_body(np_ref, dp_ref, out_ref):
    n0 = np_ref[0]
    n1 = np_ref[1]
    num = n0 + n1
    den = jnp.sum(dp_ref[...], axis=1)[:, None]
    safe = den > 0
    inv = jnp.where(safe, 1.0 / jnp.where(safe, den, 1.0), 0.0)
    out_ref[...] = num * inv


def _tc_fin(num_part, den_part):
    blk = 1000
    grid = N_NODES // blk
    return pl.pallas_call(
        _tc_fin_body,
        grid=(grid,),
        in_specs=[
            pl.BlockSpec((NC, blk, D), lambda i: (0, i, 0)),
            pl.BlockSpec((blk, NW), lambda i: (i, 0)),
        ],
        out_specs=pl.BlockSpec((blk, D), lambda i: (i, 0)),
        out_shape=jax.ShapeDtypeStruct((N_NODES, D), jnp.float32),
    )(num_part, den_part)


@jax.jit
def kernel(x, edge_index, neighborhood_values, W, a):
    row4d = edge_index[0].reshape(NW, NSA, ECA, CHUNK)
    col4d = edge_index[1].reshape(NW, NSA, ECA, CHUNK)
    nv4d = neighborhood_values.reshape(NW, NSA, ECA, CHUNK)
    a_pad = jnp.zeros((D, D), jnp.float32)
    a_pad = a_pad.at[:, 0].set(a[:D, 0]).at[:, 1].set(a[D:, 0])

    msg, alphas = _tc_pre(x, W, a_pad)
    a_src = alphas[:, 0]
    a_dst = alphas[:, 1]

    w4d, den_part = _sc_a(row4d, col4d, nv4d, a_src, a_dst)
    row4b = edge_index[0].reshape(NW, NST, SBB, CHUNK)
    col4b = edge_index[1].reshape(NW, NST, SBB, CHUNK)
    w4b = w4d.reshape(NW, NST, SBB, CHUNK)
    num_part = _sc_b(row4b, col4b, w4b, msg)

    den2d = den_part.reshape(NW, DEN_R * D)[:, :N_NODES].T
    return _tc_fin(num_part, den2d)


# speedup vs baseline: 1.2254x; 1.0280x over previous
import jax
import jax.numpy as jnp
from jax import lax
from jax.experimental import pallas as pl
from jax.experimental.pallas import tpu as pltpu
from jax.experimental.pallas import tpu_sc as plsc

N_NODES = 10000
N_EDGES = 320000
D = 128
NEG_SLOPE = 0.2

NC = 2
NS = 16
NW = NC * NS
CHUNK = 80
NCH = N_EDGES // (NW * CHUNK)
NSA = 5
ECA = NCH // NSA
SBB = 5
NST = NCH // SBB
SUP = 20
NOUT = (NCH - SBB) // SUP
ACH = N_NODES // CHUNK
DEN_R = 80


def _tc_pre_body(x_ref, w_ref, ap_ref, msg_ref, al_ref):
    msg = jnp.dot(x_ref[...], w_ref[...], preferred_element_type=jnp.float32)
    msg_ref[...] = msg
    al_ref[...] = jnp.dot(msg, ap_ref[...], preferred_element_type=jnp.float32)


def _tc_pre(x, w, a_pad):
    blk = 1000
    grid = N_NODES // blk
    return pl.pallas_call(
        _tc_pre_body,
        grid=(grid,),
        in_specs=[
            pl.BlockSpec((blk, D), lambda i: (i, 0)),
            pl.BlockSpec((D, D), lambda i: (0, 0)),
            pl.BlockSpec((D, D), lambda i: (0, 0)),
        ],
        out_specs=[
            pl.BlockSpec((blk, D), lambda i: (i, 0)),
            pl.BlockSpec((blk, D), lambda i: (i, 0)),
        ],
        out_shape=[
            jax.ShapeDtypeStruct((N_NODES, D), jnp.float32),
            jax.ShapeDtypeStruct((N_NODES, D), jnp.float32),
        ],
    )(x, w, a_pad)


def _sc_a_body(row_h, col_h, nv_h, asrc_h, adst_h, w_h, den_h,
               rowa, cola, nva, wbuf, asrc_v, adst_v, denb):
    cid = lax.axis_index("c")
    sid = lax.axis_index("s")
    wid = cid * NS + sid

    pltpu.sync_copy(asrc_h, asrc_v)
    pltpu.sync_copy(adst_h, adst_v)

    z16 = jnp.zeros((16,), jnp.float32)

    def zden(r, _):
        for k in range(D // 16):
            denb[r, pl.ds(k * 16, 16)] = z16
        return 0

    lax.fori_loop(0, DEN_R, zden, 0)

    pltpu.sync_copy(row_h.at[wid], rowa)
    pltpu.sync_copy(col_h.at[wid], cola)
    pltpu.sync_copy(nv_h.at[wid], nva)

    def chunk(j, _):
        for g in range(CHUNK // 16):
            r16 = rowa[j, pl.ds(g * 16, 16)]
            c16 = cola[j, pl.ds(g * 16, 16)]
            logit = (plsc.load_gather(asrc_v, [r16]) +
                     plsc.load_gather(adst_v, [c16]))
            logit = jnp.where(logit >= 0, logit, NEG_SLOPE * logit)
            ex = jnp.exp(logit)
            wbuf[j, pl.ds(g * 16, 16)] = nva[j, pl.ds(g * 16, 16)] * ex
            hi = lax.shift_right_logical(r16, 7)
            lo = jnp.bitwise_and(r16, 127)
            plsc.addupdate_scatter(denb, [hi, lo], ex)
        return 0

    lax.fori_loop(0, NCH, chunk, 0)
    pltpu.sync_copy(wbuf, w_h.at[wid])
    pltpu.sync_copy(denb, den_h.at[cid, sid])


def _sc_a(row4d, col4d, nv4d, a_src, a_dst):
    mesh = plsc.VectorSubcoreMesh(core_axis_name="c", subcore_axis_name="s")
    f = pl.kernel(
        _sc_a_body,
        out_type=[
            jax.ShapeDtypeStruct((NW, NCH, CHUNK), jnp.float32),
            jax.ShapeDtypeStruct((NC, NS, DEN_R, D), jnp.float32),
        ],
        mesh=mesh,
        scratch_types=[
            pltpu.VMEM((NCH, CHUNK), jnp.int32),
            pltpu.VMEM((NCH, CHUNK), jnp.int32),
            pltpu.VMEM((NCH, CHUNK), jnp.float32),
            pltpu.VMEM((NCH, CHUNK), jnp.float32),
            pltpu.VMEM((N_NODES,), jnp.float32),
            pltpu.VMEM((N_NODES,), jnp.float32),
            pltpu.VMEM((DEN_R, D), jnp.float32),
        ],
        compiler_params=pltpu.CompilerParams(needs_layout_passes=False),
    )
    return f(row4d, col4d, nv4d, a_src, a_dst)


def _sc_b_body(row_h, col_h, w_h, msg_h, num_h,
               rows0, rows1, rows2, rows3,
               ecr0, ecr1, ecc0, ecc1, ecw0, ecw1,
               w_v, acc_sh, g0, g1, g2, g3, s0, s1, s2, s3, esem):
    cid = lax.axis_index("c")
    sid = lax.axis_index("s")
    wid = cid * NS + sid
    bufs = [rows0, rows1, rows2, rows3]
    ecrs = [ecr0, ecr1]
    eccs = [ecc0, ecc1]
    ecws = [ecw0, ecw1]
    gsems = [g0, g1, g2, g3]
    ssems = [s0, s1, s2, s3]

    z16 = jnp.zeros((16,), jnp.float32)

    def zr(r, _):
        for k in range(D // 16):
            rows0[r, pl.ds(k * 16, 16)] = z16
        return 0

    lax.fori_loop(0, CHUNK, zr, 0)
    for k in range(ACH // NS + 1):
        t = sid + NS * k

        @pl.when(t < ACH)
        def _():
            pltpu.sync_copy(rows0, acc_sh.at[pl.ds(t * CHUNK, CHUNK)])

    plsc.subcore_barrier()

    pltpu.sync_copy(row_h.at[wid, 0], ecrs[0])
    pltpu.sync_copy(col_h.at[wid, 0], eccs[0])
    pltpu.sync_copy(w_h.at[wid, 0], ecws[0])

    pltpu.async_copy(msg_h.at[eccs[0].at[0]], bufs[0], gsems[0])
    pltpu.async_copy(msg_h.at[eccs[0].at[1]], bufs[1], gsems[1])

    def step(p, t):
        u = p * SUP + t
        b = t % 4
        sl = (t // SBB) % 2
        jj = t % SBB

        pltpu.make_async_copy(
            msg_h.at[pl.ds(0, CHUNK)], bufs[b], gsems[b]).wait()

        u2 = u + 2
        t2 = t + 2
        b2 = t2 % 4
        sl2 = (t2 // SBB) % 2
        jj2 = t2 % SBB

        @pl.when(u2 < NCH)
        def _():
            @pl.when(u >= 2)
            def _():
                pltpu.make_async_copy(
                    bufs[b2], acc_sh.at[pl.ds(0, CHUNK)], ssems[b2]).wait()

            if jj2 == 0:
                @pl.when(u2 >= SBB)
                def _():
                    pltpu.make_async_copy(
                        row_h.at[wid, 0], ecrs[sl2], esem).wait()
                    pltpu.make_async_copy(
                        col_h.at[wid, 0], eccs[sl2], esem).wait()
                    pltpu.make_async_copy(
                        w_h.at[wid, 0], ecws[sl2], esem).wait()

            if jj2 == 3:
                sn = lax.div(u2, SBB) + 1
                sln = (sl2 + 1) % 2

                @pl.when(sn < NST)
                def _():
                    pltpu.async_copy(row_h.at[wid, sn], ecrs[sln], esem)
                    pltpu.async_copy(col_h.at[wid, sn], eccs[sln], esem)
                    pltpu.async_copy(w_h.at[wid, sn], ecws[sln], esem)

            pltpu.async_copy(msg_h.at[eccs[sl2].at[jj2]], bufs[b2], gsems[b2])

        for g in range(CHUNK // 16):
            w_v[pl.ds(g * 16, 16)] = ecws[sl][jj, pl.ds(g * 16, 16)]

        def edge4(q, _):
            for dd in range(4):
                i = q * 4 + dd
                w = plsc.load_gather(w_v, [jnp.full((16,), i, jnp.int32)])
                for kk in range(D // 16):
                    bufs[b][i, pl.ds(kk * 16, 16)] = (
                        bufs[b][i, pl.ds(kk * 16, 16)] * w)
            return 0

        lax.fori_loop(0, CHUNK // 4, edge4, 0)

        pltpu.async_copy(bufs[b], acc_sh.at[ecrs[sl].at[jj]], ssems[b],
                         add=True)

    def outer(p, _):
        for t in range(SUP):
            step(p, t)
        return 0

    lax.fori_loop(0, NOUT, outer, 0)
    for t in range(SBB):
        step(jnp.int32(NOUT), t)

    for b in (1, 2, 3, 0):
        pltpu.make_async_copy(
            bufs[b], acc_sh.at[pl.ds(0, CHUNK)], ssems[b]).wait()

    plsc.subcore_barrier()
    for k in range(ACH // NS + 1):
        t = sid + NS * k

        @pl.when(t < ACH)
        def _():
            pltpu.sync_copy(acc_sh.at[pl.ds(t * CHUNK, CHUNK)],
                            num_h.at[cid, pl.ds(t * CHUNK, CHUNK)])


def _sc_b(row4d, col4d, w4d, msg):
    mesh = plsc.VectorSubcoreMesh(core_axis_name="c", subcore_axis_name="s")
    f = pl.kernel(
        _sc_b_body,
        out_type=jax.ShapeDtypeStruct((NC, N_NODES, D), jnp.float32),
        mesh=mesh,
        scratch_types=[
            pltpu.VMEM((CHUNK, D), jnp.float32),
            pltpu.VMEM((CHUNK, D), jnp.float32),
            pltpu.VMEM((CHUNK, D), jnp.float32),
            pltpu.VMEM((CHUNK, D), jnp.float32),
            pltpu.VMEM((SBB, CHUNK), jnp.int32),
            pltpu.VMEM((SBB, CHUNK), jnp.int32),
            pltpu.VMEM((SBB, CHUNK), jnp.int32),
            pltpu.VMEM((SBB, CHUNK), jnp.int32),
            pltpu.VMEM((SBB, CHUNK), jnp.float32),
            pltpu.VMEM((SBB, CHUNK), jnp.float32),
            pltpu.VMEM((CHUNK,), jnp.float32),
            pltpu.VMEM_SHARED((N_NODES, D), jnp.float32),
            pltpu.SemaphoreType.DMA,
            pltpu.SemaphoreType.DMA,
            pltpu.SemaphoreType.DMA,
            pltpu.SemaphoreType.DMA,
            pltpu.SemaphoreType.DMA,
            pltpu.SemaphoreType.DMA,
            pltpu.SemaphoreType.DMA,
            pltpu.SemaphoreType.DMA,
            pltpu.SemaphoreType.DMA,
        ],
        compiler_params=pltpu.CompilerParams(needs_layout_passes=False),
    )
    return f(row4d, col4d, w4d, msg)


def _tc_fin_body(np_ref, dp_ref, out_ref):
    n0 = np_ref[0]
    n1 = np_ref[1]
    num = n0 + n1
    den = jnp.sum(dp_ref[...], axis=1)[:, None]
    safe = den > 0
    inv = jnp.where(safe, 1.0 / jnp.where(safe, den, 1.0), 0.0)
    out_ref[...] = num * inv


def _tc_fin(num_part, den_part):
    blk = 1000
    grid = N_NODES // blk
    return pl.pallas_call(
        _tc_fin_body,
        grid=(grid,),
        in_specs=[
            pl.BlockSpec((NC, blk, D), lambda i: (0, i, 0)),
            pl.BlockSpec((blk, NW), lambda i: (i, 0)),
        ],
        out_specs=pl.BlockSpec((blk, D), lambda i: (i, 0)),
        out_shape=jax.ShapeDtypeStruct((N_NODES, D), jnp.float32),
    )(num_part, den_part)


@jax.jit
def kernel(x, edge_index, neighborhood_values, W, a):
    row3a = edge_index[0].reshape(NW, NCH, CHUNK)
    col3a = edge_index[1].reshape(NW, NCH, CHUNK)
    nv3a = neighborhood_values.reshape(NW, NCH, CHUNK)
    a_pad = jnp.zeros((D, D), jnp.float32)
    a_pad = a_pad.at[:, 0].set(a[:D, 0]).at[:, 1].set(a[D:, 0])

    msg, alphas = _tc_pre(x, W, a_pad)
    a_src = alphas[:, 0]
    a_dst = alphas[:, 1]

    w3d, den_part = _sc_a(row3a, col3a, nv3a, a_src, a_dst)
    row4b = edge_index[0].reshape(NW, NST, SBB, CHUNK)
    col4b = edge_index[1].reshape(NW, NST, SBB, CHUNK)
    w4b = w3d.reshape(NW, NST, SBB, CHUNK)
    num_part = _sc_b(row4b, col4b, w4b, msg)

    den2d = den_part.reshape(NW, DEN_R * D)[:, :N_NODES].T
    return _tc_fin(num_part, den2d)
